# baseline (speedup 1.0000x reference)
import functools

import jax
import jax.numpy as jnp
from jax import lax
from jax.experimental import pallas as pl
from jax.experimental.pallas import tpu as pltpu
from jax.experimental.pallas import tpu_sc as plsc

NROWS = 16384
NCOLS = 720
NOUT = 256
NLANES = 16
NC = 2
NS = 16
NW = NC * NS
RPW = NROWS // NW
RB = 64
HB = RB // 2
NB = RPW // RB
NG = NOUT // NLANES

_mesh = plsc.VectorSubcoreMesh(core_axis_name="c", subcore_axis_name="s")


@functools.partial(
    pl.kernel,
    out_type=jax.ShapeDtypeStruct((NROWS, NOUT), jnp.float32),
    mesh=_mesh,
    compiler_params=pltpu.CompilerParams(needs_layout_passes=False),
    scratch_types=[
        pltpu.VMEM((NOUT,), jnp.int32),
        pltpu.VMEM((RB, NCOLS), jnp.float32),
        pltpu.VMEM((RB, NCOLS), jnp.float32),
        pltpu.VMEM((HB, NOUT), jnp.float32),
        pltpu.VMEM((HB, NOUT), jnp.float32),
        pltpu.SemaphoreType.DMA,
        pltpu.SemaphoreType.DMA,
        pltpu.SemaphoreType.DMA,
        pltpu.SemaphoreType.DMA,
    ],
)
def _gather_k(x_hbm, idx_hbm, out_hbm, idx_v, in0, in1, o0, o1,
              si0, si1, so0, so1):
    wid = lax.axis_index("s") * NC + lax.axis_index("c")
    row0 = wid * RPW

    pltpu.sync_copy(idx_hbm, idx_v)
    idxr = [idx_v[pl.ds(NLANES * g, NLANES)] for g in range(NG)]

    ins = (in0, in1)
    outs = (o0, o1)
    sin = (si0, si1)
    sout = (so0, so1)

    def in_src(blk):
        return x_hbm.at[pl.ds(row0 + blk * RB, RB)]

    def out_dst(blk, h):
        return out_hbm.at[pl.ds(row0 + blk * RB + h * HB, HB)]

    pltpu.async_copy(in_src(0), ins[0], sin[0])

    for blk in range(NB):
        b = blk % 2
        nb = (blk + 1) % 2
        if blk + 1 < NB:
            pltpu.async_copy(in_src(blk + 1), ins[nb], sin[nb])
        pltpu.make_async_copy(in_src(blk), ins[b], sin[b]).wait()

        in_v = ins[b]
        for h in range(2):
            out_v = outs[h]
            if blk >= 1:
                pltpu.make_async_copy(out_v, out_dst(blk - 1, h),
                                      sout[h]).wait()

            @plsc.parallel_loop(0, HB, 1, unroll=4)
            def row_body(r, in_v=in_v, out_v=out_v, h=h):
                rvec = jnp.full((NLANES,), r + h * HB, dtype=jnp.int32)
                for g in range(NG):
                    val = plsc.load_gather(in_v, [rvec, idxr[g]])
                    out_v[r, pl.ds(NLANES * g, NLANES)] = val

            pltpu.async_copy(out_v, out_dst(blk, h), sout[h])

    for h in range(2):
        pltpu.make_async_copy(outs[h], out_dst(NB - 1, h), sout[h]).wait()


def kernel(input, inds_idx):
    x = input.reshape(NROWS, NCOLS)
    return _gather_k(x, inds_idx)


# speedup vs baseline: 1.5443x; 1.5443x over previous
import functools

import jax
import jax.numpy as jnp
from jax import lax
from jax.experimental import pallas as pl
from jax.experimental.pallas import tpu as pltpu
from jax.experimental.pallas import tpu_sc as plsc

NROWS = 16384
NCOLS = 720
NPAIR = 360
NOUT = 256
NLANES = 16
NC = 2
NS = 16
NW = NC * NS
RPW = NROWS // NW
RB = 128
NB = RPW // RB
NG = NOUT // NLANES

_mesh = plsc.VectorSubcoreMesh(core_axis_name="c", subcore_axis_name="s")


@functools.partial(
    pl.kernel,
    out_type=jax.ShapeDtypeStruct((NROWS, NOUT), jnp.float32),
    mesh=_mesh,
    compiler_params=pltpu.CompilerParams(needs_layout_passes=False),
    scratch_types=[
        pltpu.VMEM((NOUT,), jnp.int32),
        pltpu.VMEM((NCOLS, RB), jnp.float32),
        pltpu.VMEM((RB, NOUT), jnp.float32),
        pltpu.SemaphoreType.DMA,
        pltpu.SemaphoreType.DMA,
    ],
)
def _gather_k(xt_hbm, idx_hbm, out_hbm, idx_v, in_v, out_v, si, so):
    wid = lax.axis_index("s") * NC + lax.axis_index("c")
    row0 = wid * RPW

    pltpu.sync_copy(idx_hbm, idx_v)
    idxr = [idx_v[pl.ds(NLANES * g, NLANES)] for g in range(NG)]

    for blk in range(NB):
        i0 = row0 + blk * RB

        def dma_body(c1, carry):
            pltpu.async_copy(xt_hbm.at[c1, :, pl.ds(i0, RB)],
                             in_v.at[pl.ds(2 * c1, 2)], si)
            return carry

        lax.fori_loop(0, NPAIR, dma_body, 0)

        def wait_body(c1, carry):
            pltpu.make_async_copy(xt_hbm.at[0, :, pl.ds(i0, RB)],
                                  in_v.at[pl.ds(0, 2)], si).wait()
            return carry

        lax.fori_loop(0, NPAIR, wait_body, 0)

        if blk > 0:
            pltpu.make_async_copy(
                out_v, out_hbm.at[pl.ds(row0 + (blk - 1) * RB, RB)],
                so).wait()

        @plsc.parallel_loop(0, RB, 1, unroll=2)
        def row_body(r):
            rvec = jnp.full((NLANES,), r, dtype=jnp.int32)
            for g in range(NG):
                val = plsc.load_gather(in_v, [idxr[g], rvec])
                out_v[r, pl.ds(NLANES * g, NLANES)] = val

        pltpu.async_copy(out_v, out_hbm.at[pl.ds(i0, RB)], so)

    pltpu.make_async_copy(
        out_v, out_hbm.at[pl.ds(row0 + (NB - 1) * RB, RB)], so).wait()


def kernel(input, inds_idx):
    xt = input.transpose(1, 2, 0)
    return _gather_k(xt, inds_idx)
